# baseline (speedup 1.0000x reference)
import functools

import jax
import jax.numpy as jnp
from jax import lax
from jax.experimental import pallas as pl
from jax.experimental.pallas import tpu as pltpu
from jax.experimental.pallas import tpu_sc as plsc

NPOS = 1000
EMB_DIM = 64
BATCH = 4096
HIST = 200

NC = 2
NS = 16
NW = NC * NS

B = BATCH * HIST
B_PER_W = B // NW
BATCH_PER_W = BATCH // NW

GATHER_SPLITS = ((0, 104), (104, 96))
N_CHUNKS = BATCH_PER_W
NBUF = 2

_mesh = plsc.VectorSubcoreMesh(core_axis_name="c", subcore_axis_name="s")


@functools.partial(
    pl.kernel,
    out_type=jax.ShapeDtypeStruct((BATCH, HIST, EMB_DIM), jnp.float32),
    mesh=_mesh,
    compiler_params=pltpu.CompilerParams(use_tc_tiling_on_sc=True),
    scratch_types=[
        pltpu.VMEM((B_PER_W,), jnp.int32),
        pltpu.VMEM((NBUF, HIST, EMB_DIM), jnp.float32),
        pltpu.VMEM_SHARED((NPOS, EMB_DIM), jnp.float32),
    ] + [pltpu.SemaphoreType.DMA] * (2 * NBUF + 1),
)
def _emb_lookup(x_hbm, w_hbm, out_hbm, idx_all, rows_v, tab_sh, *sems):
    gsem = sems[:NBUF]
    ssem = sems[NBUF:2 * NBUF]
    lsem = sems[2 * NBUF]

    wid = lax.axis_index("s") * NC + lax.axis_index("c")
    base = wid * B_PER_W
    b0 = wid * BATCH_PER_W

    idx_cp = pltpu.async_copy(
        x_hbm.at[pl.ds(base, B_PER_W)], idx_all, lsem)

    @pl.when(lax.axis_index("s") == 0)
    def _():
        pltpu.sync_copy(w_hbm, tab_sh)

    idx_cp.wait()
    plsc.subcore_barrier()

    def issue_gathers(c, s):
        for off, n in GATHER_SPLITS:
            pltpu.async_copy(
                tab_sh.at[idx_all.at[pl.ds(c * HIST + off, n)]],
                rows_v.at[s, pl.ds(off, n)],
                gsem[s])

    def wait_gathers(c, s):
        for off, n in GATHER_SPLITS:
            pltpu.make_async_copy(
                tab_sh.at[idx_all.at[pl.ds(c * HIST + off, n)]],
                rows_v.at[s, pl.ds(off, n)],
                gsem[s]).wait()

    def issue_store(c, s):
        pltpu.async_copy(rows_v.at[s], out_hbm.at[b0 + c], ssem[s])

    def wait_store(c, s):
        pltpu.make_async_copy(
            rows_v.at[s], out_hbm.at[b0 + c], ssem[s]).wait()

    for s in range(NBUF):
        issue_gathers(s, s)

    @pl.loop(0, N_CHUNKS, step=NBUF)
    def _(g):
        for s in range(NBUF):
            c = g + s
            wait_gathers(c, s)
            issue_store(c, s)

            @pl.when(c + NBUF < N_CHUNKS)
            def _():
                wait_store(c, s)
                issue_gathers(c + NBUF, s)

    for s in range(NBUF):
        wait_store(N_CHUNKS - NBUF + s, s)


def kernel(x, pos_emb_weight):
    w = pos_emb_weight.at[0].set(0.0)
    x_flat = x.astype(jnp.int32).reshape(B)
    return _emb_lookup(x_flat, w)
